# Initial kernel scaffold; baseline (speedup 1.0000x reference)
import jax
import jax.numpy as jnp
from jax.experimental import pallas as pl


def kernel(x, adj, edge_index, batch, W1, b1, W2, b2):
    raise NotImplementedError("write your pallas kernel here")


# speedup vs baseline: 11.0463x; 11.0463x over previous
import jax
import jax.numpy as jnp
from jax import lax
from jax.experimental import pallas as pl
from jax.experimental.pallas import tpu as pltpu
from jax.experimental.pallas import tpu_sc as plsc

N = 900
NN = N * N
NG = 10
NPG = 90
F1 = 90
E = 28800

NC = 2
NS = 16
NW = NC * NS
EPT = 1024
E2 = EPT * NW
CH = EPT // 128
PT_A = 50640
ASZ = NS * PT_A


def _adj_body(edges, out, src_v, dst_v, flat_v, ones_v, zero_v, acc):
    c = lax.axis_index("c")
    s = lax.axis_index("s")
    wid = c * NS + s

    def fill_zero(i, _):
        zero_v[pl.ds(i * 16, 16)] = jnp.zeros((16,), jnp.float32)
        return 0

    lax.fori_loop(0, PT_A // 16, fill_zero, 0)
    for k in range(128 // 16):
        ones_v[pl.ds(k * 16, 16)] = jnp.full((16,), 1.0, jnp.float32)

    pltpu.sync_copy(zero_v, acc.at[pl.ds(s * PT_A, PT_A)])

    base = wid * EPT
    pltpu.sync_copy(edges.at[pl.ds(base, EPT)], src_v)
    pltpu.sync_copy(edges.at[pl.ds(E2 + base, EPT)], dst_v)

    for j in range(CH):
        def fill_flat(k, _, j=j):
            off = j * 128 + k * 16
            fl = dst_v[pl.ds(off, 16)] * N + src_v[pl.ds(off, 16)]
            flat_v[j, pl.ds(k * 16, 16)] = fl
            return 0

        lax.fori_loop(0, 128 // 16, fill_flat, 0)

    plsc.subcore_barrier()
    for j in range(CH):
        pltpu.sync_copy(ones_v, acc.at[flat_v.at[j]], add=True)
    plsc.subcore_barrier()

    pltpu.sync_copy(acc.at[pl.ds(s * PT_A, PT_A)], zero_v)
    pltpu.sync_copy(zero_v, out.at[pl.ds(c * ASZ + s * PT_A, PT_A)])


def _adj_call(ei):
    k = pl.kernel(
        _adj_body,
        out_type=jax.ShapeDtypeStruct((NC * ASZ,), jnp.float32),
        mesh=plsc.VectorSubcoreMesh(core_axis_name="c", subcore_axis_name="s",
                                    num_cores=NC),
        scratch_types=[
            pltpu.VMEM((EPT,), jnp.int32),
            pltpu.VMEM((EPT,), jnp.int32),
            pltpu.VMEM((CH, 128), jnp.int32),
            pltpu.VMEM((128,), jnp.float32),
            pltpu.VMEM((PT_A,), jnp.float32),
            pltpu.VMEM_SHARED((ASZ,), jnp.float32),
        ],
    )
    return k(ei)


def _dense_body(p_ref, x_ref, batch_ref, w1_ref, b1_ref, w2_ref, b2_ref,
                out_ref):
    At = p_ref[0] + p_ref[1]
    ri = lax.broadcasted_iota(jnp.int32, (N, N), 0)
    ci = lax.broadcasted_iota(jnp.int32, (N, N), 1)
    At = At + jnp.where(ri == ci, 1.0, 0.0)

    deg = jnp.sum(At, axis=1, keepdims=True)
    dinv = lax.rsqrt(deg)

    g1 = dinv * jnp.dot(x_ref[:], w1_ref[:], preferred_element_type=jnp.float32)
    h1 = dinv * jnp.dot(At, g1, preferred_element_type=jnp.float32) + b1_ref[:]
    h1 = jnp.maximum(h1, 0.0)

    ssum = h1[0:NPG, :]
    for b in range(1, NG):
        ssum = ssum + h1[b * NPG:(b + 1) * NPG, :]
    sm = ssum * (1.0 / NG)
    sm = sm - jnp.max(sm, axis=1, keepdims=True)
    es = jnp.exp(sm)
    s_soft = es / jnp.sum(es, axis=1, keepdims=True)

    blocks = []
    for b in range(NG):
        hb = h1[b * NPG:(b + 1) * NPG, :]
        blocks.append(
            lax.dot_general(s_soft, hb, (((0,), (0,)), ((), ())),
                            preferred_element_type=jnp.float32))
    h2 = jnp.concatenate(blocks, axis=0)

    g2 = dinv * jnp.dot(h2, w2_ref[:], preferred_element_type=jnp.float32)
    h3 = dinv * jnp.dot(At, g2, preferred_element_type=jnp.float32) + b2_ref[:]

    gi = lax.broadcasted_iota(jnp.int32, (NG, N), 0)
    bmat = jnp.where(batch_ref[:] == gi, 1.0, 0.0)
    cnt = jnp.sum(bmat, axis=1, keepdims=True)
    gm = jnp.dot(bmat, h3, preferred_element_type=jnp.float32)
    gm = gm / jnp.maximum(cnt, 1.0)

    z = gm - jnp.max(gm, axis=1, keepdims=True)
    out_ref[:] = z - jnp.log(jnp.sum(jnp.exp(z), axis=1, keepdims=True))


def _dense_call(P, x, batch2d, W1, b1, W2, b2):
    return pl.pallas_call(
        _dense_body,
        out_shape=jax.ShapeDtypeStruct((NG, 4), jnp.float32),
    )(P, x, batch2d, W1, b1, W2, b2)


def kernel(x, adj, edge_index, batch, W1, b1, W2, b2):
    del adj
    pad = jnp.broadcast_to(jnp.array([[0], [N]], jnp.int32), (2, E2 - E))
    ei = jnp.concatenate([edge_index.astype(jnp.int32), pad], axis=1).reshape(-1)
    parts = _adj_call(ei).reshape(NC, ASZ)
    P = parts[:, :NN].reshape(NC, N, N)
    return _dense_call(P, x, batch.astype(jnp.int32).reshape(1, N), W1,
                       b1.reshape(1, F1), W2, b2.reshape(1, 4))


# speedup vs baseline: 16.8447x; 1.5249x over previous
import jax
import jax.numpy as jnp
from jax import lax
from jax.experimental import pallas as pl
from jax.experimental.pallas import tpu as pltpu
from jax.experimental.pallas import tpu_sc as plsc

N = 900
NN = N * N
NG = 10
NPG = 90
F1 = 90
E = 28800

NS = 16
EPT = 1920
E2 = EPT * NS
CH = EPT // 128
PT_A = 50640
PT_STRIDE = 50624


def _adj_body(edges, out, src_v, dst_v, flat_v, ones_v, zero_v, acc):
    s = lax.axis_index("s")

    def fill_zero(i, _):
        zero_v[pl.ds(i * 16, 16)] = jnp.zeros((16,), jnp.float32)
        return 0

    lax.fori_loop(0, PT_A // 16, fill_zero, 0)
    for k in range(128 // 16):
        ones_v[pl.ds(k * 16, 16)] = jnp.full((16,), 1.0, jnp.float32)

    pltpu.sync_copy(zero_v, acc.at[pl.ds(s * PT_STRIDE, PT_A)])

    base = s * EPT
    pltpu.sync_copy(edges.at[pl.ds(base, EPT)], src_v)
    pltpu.sync_copy(edges.at[pl.ds(E2 + base, EPT)], dst_v)

    for j in range(CH):
        def fill_flat(k, _, j=j):
            off = j * 128 + k * 16
            fl = dst_v[pl.ds(off, 16)] * N + src_v[pl.ds(off, 16)]
            flat_v[j, pl.ds(k * 16, 16)] = fl
            return 0

        lax.fori_loop(0, 128 // 16, fill_flat, 0)

    plsc.subcore_barrier()
    for j in range(CH):
        pltpu.sync_copy(ones_v, acc.at[flat_v.at[j]], add=True)
    plsc.subcore_barrier()

    pltpu.sync_copy(acc.at[pl.ds(s * PT_STRIDE, PT_A)], zero_v)
    pltpu.sync_copy(zero_v, out.at[pl.ds(s * PT_STRIDE, PT_A)])


def _adj_call(ei):
    k = pl.kernel(
        _adj_body,
        out_type=jax.ShapeDtypeStruct((NN,), jnp.float32),
        mesh=plsc.VectorSubcoreMesh(core_axis_name="c", subcore_axis_name="s",
                                    num_cores=1),
        scratch_types=[
            pltpu.VMEM((EPT,), jnp.int32),
            pltpu.VMEM((EPT,), jnp.int32),
            pltpu.VMEM((CH, 128), jnp.int32),
            pltpu.VMEM((128,), jnp.float32),
            pltpu.VMEM((PT_A,), jnp.float32),
            pltpu.VMEM_SHARED((NN,), jnp.float32),
        ],
    )
    return k(ei)


def _dense_body(p_ref, x_ref, batch_ref, w1_ref, b1_ref, w2_ref, b2_ref,
                out_ref):
    ri = lax.broadcasted_iota(jnp.int32, (N, N), 0)
    ci = lax.broadcasted_iota(jnp.int32, (N, N), 1)
    At = p_ref[:] + jnp.where(ri == ci, 1.0, 0.0)
    At = At - jnp.where((ri == 0) & (ci == 0), float(E2 - E), 0.0)

    deg = jnp.sum(At, axis=1, keepdims=True)
    dinv = lax.rsqrt(deg)

    g1 = dinv * jnp.dot(x_ref[:], w1_ref[:], preferred_element_type=jnp.float32)
    h1 = dinv * jnp.dot(At, g1, preferred_element_type=jnp.float32) + b1_ref[:]
    h1 = jnp.maximum(h1, 0.0)

    ssum = h1[0:NPG, :]
    for b in range(1, NG):
        ssum = ssum + h1[b * NPG:(b + 1) * NPG, :]
    sm = ssum * (1.0 / NG)
    sm = sm - jnp.max(sm, axis=1, keepdims=True)
    es = jnp.exp(sm)
    s_soft = es / jnp.sum(es, axis=1, keepdims=True)

    blocks = []
    for b in range(NG):
        hb = h1[b * NPG:(b + 1) * NPG, :]
        blocks.append(
            lax.dot_general(s_soft, hb, (((0,), (0,)), ((), ())),
                            preferred_element_type=jnp.float32))
    h2 = jnp.concatenate(blocks, axis=0)

    g2 = dinv * jnp.dot(h2, w2_ref[:], preferred_element_type=jnp.float32)
    h3 = dinv * jnp.dot(At, g2, preferred_element_type=jnp.float32) + b2_ref[:]

    gi = lax.broadcasted_iota(jnp.int32, (NG, N), 0)
    bmat = jnp.where(batch_ref[:] == gi, 1.0, 0.0)
    cnt = jnp.sum(bmat, axis=1, keepdims=True)
    gm = jnp.dot(bmat, h3, preferred_element_type=jnp.float32)
    gm = gm / jnp.maximum(cnt, 1.0)

    z = gm - jnp.max(gm, axis=1, keepdims=True)
    out_ref[:] = z - jnp.log(jnp.sum(jnp.exp(z), axis=1, keepdims=True))


def _dense_call(P, x, batch2d, W1, b1, W2, b2):
    return pl.pallas_call(
        _dense_body,
        out_shape=jax.ShapeDtypeStruct((NG, 4), jnp.float32),
    )(P, x, batch2d, W1, b1, W2, b2)


def kernel(x, adj, edge_index, batch, W1, b1, W2, b2):
    del adj
    pad = jnp.zeros((2, E2 - E), jnp.int32)
    ei = jnp.concatenate([edge_index.astype(jnp.int32), pad], axis=1).reshape(-1)
    P = _adj_call(ei).reshape(N, N)
    return _dense_call(P, x, batch.astype(jnp.int32).reshape(1, N), W1,
                       b1.reshape(1, F1), W2, b2.reshape(1, 4))


# speedup vs baseline: 22.5038x; 1.3360x over previous
import jax
import jax.numpy as jnp
from jax import lax
from jax.experimental import pallas as pl
from jax.experimental.pallas import tpu as pltpu
from jax.experimental.pallas import tpu_sc as plsc

N = 900
NN = N * N
NG = 10
NPG = 90
F1 = 90
E = 28800

NS = 16
EPT = 1800
EBUF = 1808
SLOTS = 1920
E2 = SLOTS * NS
CH = SLOTS // 128
PT_A = 50640
PT_STRIDE = 50624
ZCH = 4096
RCH = 16880


def _adj_body(edges, out, src_v, dst_v, flat_v, ones_v, zero_v, acc,
              semz, seme, semr):
    s = lax.axis_index("s")

    def fill_zero(i, _):
        zero_v[pl.ds(i * 16, 16)] = jnp.zeros((16,), jnp.float32)
        return 0

    lax.fori_loop(0, ZCH // 16, fill_zero, 0)
    for k in range(128 // 16):
        ones_v[pl.ds(k * 16, 16)] = jnp.full((16,), 1.0, jnp.float32)

    zh = []
    for q in range(12):
        zh.append(pltpu.async_copy(
            zero_v.at[pl.ds(0, ZCH)],
            acc.at[pl.ds(s * PT_STRIDE + q * ZCH, ZCH)], semz))
    zh.append(pltpu.async_copy(
        zero_v.at[pl.ds(0, PT_A - 12 * ZCH)],
        acc.at[pl.ds(s * PT_STRIDE + 12 * ZCH, PT_A - 12 * ZCH)], semz))

    base = s * EPT
    eh0 = pltpu.async_copy(edges.at[pl.ds(base, EPT)],
                           src_v.at[pl.ds(0, EPT)], seme)
    eh1 = pltpu.async_copy(edges.at[pl.ds(E + base, EPT)],
                           dst_v.at[pl.ds(0, EPT)], seme)
    eh0.wait()
    eh1.wait()

    nfull = EPT // 16
    for j in range(CH):
        def fill_flat(k, _, j=j):
            off = j * 128 + k * 16
            fl = dst_v[pl.ds(off, 16)] * N + src_v[pl.ds(off, 16)]
            flat_v[j, pl.ds(k * 16, 16)] = fl
            return 0

        def fill_pad(k, _, j=j):
            flat_v[j, pl.ds(k * 16, 16)] = jnp.zeros((16,), jnp.int32)
            return 0

        lo = j * 8
        if (j + 1) * 8 <= nfull:
            lax.fori_loop(0, 8, fill_flat, 0)
        else:
            for k in range(8):
                g = lo + k
                if g < nfull:
                    fill_flat(k, 0)
                elif g == nfull:
                    off = g * 16
                    fl = dst_v[pl.ds(off, 16)] * N + src_v[pl.ds(off, 16)]
                    lane = lax.iota(jnp.int32, 16)
                    fl = jnp.where(lane < EPT - nfull * 16, fl, 0)
                    flat_v[j, pl.ds(k * 16, 16)] = fl
                else:
                    fill_pad(k, 0)

    for h in zh:
        h.wait()
    plsc.subcore_barrier()
    for j in range(CH):
        pltpu.sync_copy(ones_v, acc.at[flat_v.at[j]], add=True)
    plsc.subcore_barrier()

    rh = []
    for c in range(3):
        pltpu.sync_copy(acc.at[pl.ds(s * PT_STRIDE + c * RCH, RCH)],
                        zero_v.at[pl.ds(c * RCH, RCH)])
        rh.append(pltpu.async_copy(
            zero_v.at[pl.ds(c * RCH, RCH)],
            out.at[pl.ds(s * PT_STRIDE + c * RCH, RCH)], semr))
    for h in rh:
        h.wait()


def _adj_call(ei):
    k = pl.kernel(
        _adj_body,
        out_type=jax.ShapeDtypeStruct((NN,), jnp.float32),
        mesh=plsc.VectorSubcoreMesh(core_axis_name="c", subcore_axis_name="s",
                                    num_cores=1),
        scratch_types=[
            pltpu.VMEM((EBUF,), jnp.int32),
            pltpu.VMEM((EBUF,), jnp.int32),
            pltpu.VMEM((CH, 128), jnp.int32),
            pltpu.VMEM((128,), jnp.float32),
            pltpu.VMEM((PT_A,), jnp.float32),
            pltpu.VMEM_SHARED((NN,), jnp.float32),
            pltpu.SemaphoreType.DMA,
            pltpu.SemaphoreType.DMA,
            pltpu.SemaphoreType.DMA,
        ],
    )
    return k(ei)


def _dense_body(p_ref, x_ref, batch_ref, w1_ref, b1_ref, w2_ref, b2_ref,
                out_ref):
    ri = lax.broadcasted_iota(jnp.int32, (N, N), 0)
    ci = lax.broadcasted_iota(jnp.int32, (N, N), 1)
    At = p_ref[:] + jnp.where(ri == ci, 1.0, 0.0)
    At = At - jnp.where((ri == 0) & (ci == 0), float(E2 - E), 0.0)

    deg = jnp.sum(At, axis=1, keepdims=True)
    dinv = lax.rsqrt(deg)

    g1 = dinv * jnp.dot(x_ref[:], w1_ref[:], preferred_element_type=jnp.float32)
    h1 = dinv * jnp.dot(At, g1, preferred_element_type=jnp.float32) + b1_ref[:]
    h1 = jnp.maximum(h1, 0.0)

    ssum = h1[0:NPG, :]
    for b in range(1, NG):
        ssum = ssum + h1[b * NPG:(b + 1) * NPG, :]
    sm = ssum * (1.0 / NG)
    sm = sm - jnp.max(sm, axis=1, keepdims=True)
    es = jnp.exp(sm)
    s_soft = es / jnp.sum(es, axis=1, keepdims=True)

    blocks = []
    for b in range(NG):
        hb = h1[b * NPG:(b + 1) * NPG, :]
        blocks.append(
            lax.dot_general(s_soft, hb, (((0,), (0,)), ((), ())),
                            preferred_element_type=jnp.float32))
    h2 = jnp.concatenate(blocks, axis=0)

    g2 = dinv * jnp.dot(h2, w2_ref[:], preferred_element_type=jnp.float32)
    h3 = dinv * jnp.dot(At, g2, preferred_element_type=jnp.float32) + b2_ref[:]

    gi = lax.broadcasted_iota(jnp.int32, (NG, N), 0)
    bmat = jnp.where(batch_ref[:] == gi, 1.0, 0.0)
    cnt = jnp.sum(bmat, axis=1, keepdims=True)
    gm = jnp.dot(bmat, h3, preferred_element_type=jnp.float32)
    gm = gm / jnp.maximum(cnt, 1.0)

    z = gm - jnp.max(gm, axis=1, keepdims=True)
    out_ref[:] = z - jnp.log(jnp.sum(jnp.exp(z), axis=1, keepdims=True))


def _dense_call(P, x, batch2d, W1, b1, W2, b2):
    return pl.pallas_call(
        _dense_body,
        out_shape=jax.ShapeDtypeStruct((NG, 4), jnp.float32),
    )(P, x, batch2d, W1, b1, W2, b2)


def kernel(x, adj, edge_index, batch, W1, b1, W2, b2):
    del adj
    ei = edge_index.astype(jnp.int32).reshape(-1)
    P = _adj_call(ei).reshape(N, N)
    return _dense_call(P, x, batch.astype(jnp.int32).reshape(1, N), W1,
                       b1.reshape(1, F1), W2, b2.reshape(1, 4))


# speedup vs baseline: 22.9621x; 1.0204x over previous
import jax
import jax.numpy as jnp
from jax import lax
from jax.experimental import pallas as pl
from jax.experimental.pallas import tpu as pltpu
from jax.experimental.pallas import tpu_sc as plsc

N = 900
NN = N * N
NG = 10
NPG = 90
F1 = 90
E = 28800

NS = 16
EPT = 1800
EBUF = 1808
SLOTS = 1920
E2 = SLOTS * NS
CH = SLOTS // 128
PT_A = 50640
PT_STRIDE = 50624
ZCH = 4096
RCH = 16880


def _adj_body(edges, out, src_v, dst_v, flat_v, ones_v, zero_v, acc,
              semz, seme, semr):
    s = lax.axis_index("s")

    base = s * EPT
    eh0 = pltpu.async_copy(edges.at[pl.ds(base, EPT)],
                           src_v.at[pl.ds(0, EPT)], seme)
    eh1 = pltpu.async_copy(edges.at[pl.ds(E + base, EPT)],
                           dst_v.at[pl.ds(0, EPT)], seme)

    def fill_zero(i, _):
        zero_v[pl.ds(i * 16, 16)] = jnp.zeros((16,), jnp.float32)
        return 0

    lax.fori_loop(0, ZCH // 16, fill_zero, 0)
    for k in range(128 // 16):
        ones_v[pl.ds(k * 16, 16)] = jnp.full((16,), 1.0, jnp.float32)

    zh = []
    for q in range(12):
        zh.append(pltpu.async_copy(
            zero_v.at[pl.ds(0, ZCH)],
            acc.at[pl.ds(s * PT_STRIDE + q * ZCH, ZCH)], semz))
    zh.append(pltpu.async_copy(
        zero_v.at[pl.ds(0, PT_A - 12 * ZCH)],
        acc.at[pl.ds(s * PT_STRIDE + 12 * ZCH, PT_A - 12 * ZCH)], semz))

    eh0.wait()
    eh1.wait()

    nfull = EPT // 16
    for j in range(CH):
        def fill_flat(k, _, j=j):
            off = j * 128 + k * 16
            fl = dst_v[pl.ds(off, 16)] * N + src_v[pl.ds(off, 16)]
            flat_v[j, pl.ds(k * 16, 16)] = fl
            return 0

        def fill_pad(k, _, j=j):
            flat_v[j, pl.ds(k * 16, 16)] = jnp.zeros((16,), jnp.int32)
            return 0

        lo = j * 8
        if (j + 1) * 8 <= nfull:
            lax.fori_loop(0, 8, fill_flat, 0)
        else:
            for k in range(8):
                g = lo + k
                if g < nfull:
                    fill_flat(k, 0)
                elif g == nfull:
                    off = g * 16
                    fl = dst_v[pl.ds(off, 16)] * N + src_v[pl.ds(off, 16)]
                    lane = lax.iota(jnp.int32, 16)
                    fl = jnp.where(lane < EPT - nfull * 16, fl, 0)
                    flat_v[j, pl.ds(k * 16, 16)] = fl
                else:
                    fill_pad(k, 0)

    for h in zh:
        h.wait()
    plsc.subcore_barrier()
    sh = []
    for j in range(CH):
        sh.append(pltpu.async_copy(ones_v, acc.at[flat_v.at[j]], seme,
                                   add=True))
    for h in sh:
        h.wait()
    plsc.subcore_barrier()

    rh = []
    for c in range(3):
        pltpu.sync_copy(acc.at[pl.ds(s * PT_STRIDE + c * RCH, RCH)],
                        zero_v.at[pl.ds(c * RCH, RCH)])
        rh.append(pltpu.async_copy(
            zero_v.at[pl.ds(c * RCH, RCH)],
            out.at[pl.ds(s * PT_STRIDE + c * RCH, RCH)], semr))
    for h in rh:
        h.wait()


def _adj_call(ei):
    k = pl.kernel(
        _adj_body,
        out_type=jax.ShapeDtypeStruct((NN,), jnp.float32),
        mesh=plsc.VectorSubcoreMesh(core_axis_name="c", subcore_axis_name="s",
                                    num_cores=1),
        scratch_types=[
            pltpu.VMEM((EBUF,), jnp.int32),
            pltpu.VMEM((EBUF,), jnp.int32),
            pltpu.VMEM((CH, 128), jnp.int32),
            pltpu.VMEM((128,), jnp.float32),
            pltpu.VMEM((PT_A,), jnp.float32),
            pltpu.VMEM_SHARED((NN,), jnp.float32),
            pltpu.SemaphoreType.DMA,
            pltpu.SemaphoreType.DMA,
            pltpu.SemaphoreType.DMA,
        ],
    )
    return k(ei)


def _dense_body(p_ref, x_ref, batch_ref, w1_ref, b1_ref, w2_ref, b2_ref,
                out_ref):
    ri = lax.broadcasted_iota(jnp.int32, (N, N), 0)
    ci = lax.broadcasted_iota(jnp.int32, (N, N), 1)
    At = p_ref[:] + jnp.where(ri == ci, 1.0, 0.0)
    At = At - jnp.where((ri == 0) & (ci == 0), float(E2 - E), 0.0)

    deg = jnp.sum(At, axis=1, keepdims=True)
    dinv = lax.rsqrt(deg)

    g1 = dinv * jnp.dot(x_ref[:], w1_ref[:], preferred_element_type=jnp.float32)
    h1 = dinv * jnp.dot(At, g1, preferred_element_type=jnp.float32) + b1_ref[:]
    h1 = jnp.maximum(h1, 0.0)

    ssum = h1[0:NPG, :]
    for b in range(1, NG):
        ssum = ssum + h1[b * NPG:(b + 1) * NPG, :]
    sm = ssum * (1.0 / NG)
    sm = sm - jnp.max(sm, axis=1, keepdims=True)
    es = jnp.exp(sm)
    s_soft = es / jnp.sum(es, axis=1, keepdims=True)

    blocks = []
    for b in range(NG):
        hb = h1[b * NPG:(b + 1) * NPG, :]
        blocks.append(
            lax.dot_general(s_soft, hb, (((0,), (0,)), ((), ())),
                            preferred_element_type=jnp.float32))
    h2 = jnp.concatenate(blocks, axis=0)

    g2 = dinv * jnp.dot(h2, w2_ref[:], preferred_element_type=jnp.float32)
    h3 = dinv * jnp.dot(At, g2, preferred_element_type=jnp.float32) + b2_ref[:]

    gi = lax.broadcasted_iota(jnp.int32, (NG, N), 0)
    bmat = jnp.where(batch_ref[:] == gi, 1.0, 0.0)
    cnt = jnp.sum(bmat, axis=1, keepdims=True)
    gm = jnp.dot(bmat, h3, preferred_element_type=jnp.float32)
    gm = gm / jnp.maximum(cnt, 1.0)

    z = gm - jnp.max(gm, axis=1, keepdims=True)
    out_ref[:] = z - jnp.log(jnp.sum(jnp.exp(z), axis=1, keepdims=True))


def _dense_call(P, x, batch2d, W1, b1, W2, b2):
    return pl.pallas_call(
        _dense_body,
        out_shape=jax.ShapeDtypeStruct((NG, 4), jnp.float32),
    )(P, x, batch2d, W1, b1, W2, b2)


def kernel(x, adj, edge_index, batch, W1, b1, W2, b2):
    del adj
    ei = edge_index.astype(jnp.int32).reshape(-1)
    P = _adj_call(ei).reshape(N, N)
    return _dense_call(P, x, batch.astype(jnp.int32).reshape(1, N), W1,
                       b1.reshape(1, F1), W2, b2.reshape(1, 4))
